# Initial kernel scaffold; baseline (speedup 1.0000x reference)
import jax
import jax.numpy as jnp
from jax.experimental import pallas as pl


def kernel(features, edge_index, W1, b1, W2, b2, W3, b3):
    raise NotImplementedError("write your pallas kernel here")


# speedup vs baseline: 19.6234x; 19.6234x over previous
import functools

import jax
import jax.numpy as jnp
from jax import lax
from jax.experimental import pallas as pl
from jax.experimental.pallas import tpu as pltpu
from jax.experimental.pallas import tpu_sc as plsc

N = 10000
E = 320000
D_IN = 128
D_H = 128
D_OUT = 64

NC = 2
NS = 16
NT = NC * NS
CHUNK = 128
CPT = 80
IB = 16
E_PAD = NT * CPT * CHUNK
N_PAD = 10240
RPT = N_PAD // NS
DEG_W = 128
BR = 2000

_mesh = plsc.VectorSubcoreMesh(
    core_axis_name="c", subcore_axis_name="s", num_cores=NC, num_subcores=NS)


@functools.partial(
    pl.kernel,
    out_type=jax.ShapeDtypeStruct((NC, N_PAD, DEG_W), jnp.float32),
    mesh=_mesh,
    scratch_types=[
        pltpu.VMEM((IB, CHUNK), jnp.int32),
        pltpu.VMEM((CHUNK, DEG_W), jnp.float32),
        pltpu.VMEM_SHARED((N_PAD, DEG_W), jnp.float32),
    ],
)
def _sc_deg(dst_hbm, out_hbm, dst_v, buf_v, acc_sh):
    c = lax.axis_index("c")
    s = lax.axis_index("s")
    t = c * NS + s
    zero16 = jnp.zeros((16,), jnp.float32)
    one16 = jnp.full((16,), 1.0, jnp.float32)

    @pl.loop(0, CHUNK)
    def _zero(r):
        for k in range(DEG_W // 16):
            buf_v[r, pl.ds(k * 16, 16)] = zero16

    @pl.loop(0, RPT // CHUNK)
    def _init(k):
        pltpu.sync_copy(buf_v, acc_sh.at[pl.ds(s * RPT + k * CHUNK, CHUNK)])

    @pl.loop(0, CHUNK)
    def _ones(r):
        for k in range(DEG_W // 16):
            buf_v[r, pl.ds(k * 16, 16)] = one16

    plsc.subcore_barrier()

    @pl.loop(0, CPT // IB)
    def _blk(bi):
        pltpu.sync_copy(dst_hbm.at[pl.ds(t * CPT + bi * IB, IB)], dst_v)

        @pl.loop(0, IB)
        def _scatter(j):
            pltpu.sync_copy(buf_v, acc_sh.at[dst_v.at[j]], add=True)

    plsc.subcore_barrier()

    @pl.loop(0, RPT // CHUNK)
    def _wb(k):
        off = s * RPT + k * CHUNK
        pltpu.sync_copy(acc_sh.at[pl.ds(off, CHUNK)], buf_v)
        pltpu.sync_copy(buf_v, out_hbm.at[c, pl.ds(off, CHUNK)])


def _make_sc_agg(D):
    @functools.partial(
        pl.kernel,
        out_type=jax.ShapeDtypeStruct((NC, N_PAD, D), jnp.float32),
        mesh=_mesh,
        scratch_types=[
            pltpu.VMEM((IB, CHUNK), jnp.int32),
            pltpu.VMEM((IB, CHUNK), jnp.int32),
            pltpu.VMEM((CHUNK, D), jnp.float32),
            pltpu.VMEM((CHUNK, D), jnp.float32),
            pltpu.VMEM_SHARED((N_PAD, D), jnp.float32),
            pltpu.SemaphoreType.DMA,
            pltpu.SemaphoreType.DMA,
        ],
    )
    def _sc_agg(z_hbm, src_hbm, dst_hbm, out_hbm,
                src_v, dst_v, ra, rb, acc_sh, sa, sb):
        c = lax.axis_index("c")
        s = lax.axis_index("s")
        t = c * NS + s
        zero16 = jnp.zeros((16,), jnp.float32)

        @pl.loop(0, CHUNK)
        def _zero(r):
            for k in range(D // 16):
                ra[r, pl.ds(k * 16, 16)] = zero16

        @pl.loop(0, RPT // CHUNK)
        def _init(k):
            pltpu.sync_copy(ra, acc_sh.at[pl.ds(s * RPT + k * CHUNK, CHUNK)])

        plsc.subcore_barrier()

        @pl.loop(0, CPT // IB)
        def _blk(bi):
            base = t * CPT + bi * IB
            pltpu.sync_copy(src_hbm.at[pl.ds(base, IB)], src_v)
            pltpu.sync_copy(dst_hbm.at[pl.ds(base, IB)], dst_v)

            @pl.loop(0, IB, step=2)
            def _edges(j):
                ca = pltpu.async_copy(z_hbm.at[src_v.at[j]], ra, sa)
                cb = pltpu.async_copy(z_hbm.at[src_v.at[j + 1]], rb, sb)
                ca.wait()
                pltpu.sync_copy(ra, acc_sh.at[dst_v.at[j]], add=True)
                cb.wait()
                pltpu.sync_copy(rb, acc_sh.at[dst_v.at[j + 1]], add=True)

        plsc.subcore_barrier()

        @pl.loop(0, RPT // CHUNK)
        def _wb(k):
            off = s * RPT + k * CHUNK
            pltpu.sync_copy(acc_sh.at[pl.ds(off, CHUNK)], ra)
            pltpu.sync_copy(ra, out_hbm.at[c, pl.ds(off, CHUNK)])

    return _sc_agg


_sc_agg_h = _make_sc_agg(D_H)


def _tc_first_body(degp_ref, x_ref, w_ref, dinv_ref, z_ref):
    deg = degp_ref[0, :, 0:1] + degp_ref[1, :, 0:1] + 1.0
    dinv = lax.rsqrt(deg)
    dinv_ref[...] = dinv
    xw = jnp.dot(x_ref[...], w_ref[...], preferred_element_type=jnp.float32)
    z_ref[...] = xw * dinv


def _tc_first(deg_parts, features, W1):
    return pl.pallas_call(
        _tc_first_body,
        grid=(N // BR,),
        in_specs=[
            pl.BlockSpec((NC, BR, DEG_W), lambda i: (0, i, 0)),
            pl.BlockSpec((BR, D_IN), lambda i: (i, 0)),
            pl.BlockSpec((D_IN, D_H), lambda i: (0, 0)),
        ],
        out_specs=[
            pl.BlockSpec((BR, 1), lambda i: (i, 0)),
            pl.BlockSpec((BR, D_H), lambda i: (i, 0)),
        ],
        out_shape=[
            jax.ShapeDtypeStruct((N, 1), jnp.float32),
            jax.ShapeDtypeStruct((N, D_H), jnp.float32),
        ],
    )(deg_parts, features, W1)


def _tc_mid_body(acc_ref, z_ref, dinv_ref, b_ref, w_ref, zn_ref):
    dinv = dinv_ref[...]
    x = (acc_ref[0] + acc_ref[1] + z_ref[...]) * dinv + b_ref[...]
    x = jnp.maximum(x, 0.0)
    zn_ref[...] = jnp.dot(x, w_ref[...], preferred_element_type=jnp.float32) * dinv


def _tc_mid(acc_parts, z, dinv, b, W, d_out):
    d_in = z.shape[1]
    return pl.pallas_call(
        _tc_mid_body,
        grid=(N // BR,),
        in_specs=[
            pl.BlockSpec((NC, BR, d_in), lambda i: (0, i, 0)),
            pl.BlockSpec((BR, d_in), lambda i: (i, 0)),
            pl.BlockSpec((BR, 1), lambda i: (i, 0)),
            pl.BlockSpec((1, d_in), lambda i: (0, 0)),
            pl.BlockSpec((d_in, d_out), lambda i: (0, 0)),
        ],
        out_specs=pl.BlockSpec((BR, d_out), lambda i: (i, 0)),
        out_shape=jax.ShapeDtypeStruct((N, d_out), jnp.float32),
    )(acc_parts, z, dinv, b, W)


def _tc_last_body(acc_ref, z_ref, dinv_ref, b_ref, out_ref):
    acc = acc_ref[0, :, 0:D_OUT] + acc_ref[1, :, 0:D_OUT]
    x = (acc + z_ref[:, 0:D_OUT]) * dinv_ref[...] + b_ref[...]
    x = jnp.maximum(x, 0.0)
    m = jnp.max(x, axis=-1, keepdims=True)
    lse = jnp.log(jnp.sum(jnp.exp(x - m), axis=-1, keepdims=True)) + m
    out_ref[...] = x - lse


def _tc_last(acc_parts, z, dinv, b):
    return pl.pallas_call(
        _tc_last_body,
        grid=(N // BR,),
        in_specs=[
            pl.BlockSpec((NC, BR, D_H), lambda i: (0, i, 0)),
            pl.BlockSpec((BR, D_H), lambda i: (i, 0)),
            pl.BlockSpec((BR, 1), lambda i: (i, 0)),
            pl.BlockSpec((1, D_OUT), lambda i: (0, 0)),
        ],
        out_specs=pl.BlockSpec((BR, D_OUT), lambda i: (i, 0)),
        out_shape=jax.ShapeDtypeStruct((N, D_OUT), jnp.float32),
    )(acc_parts, z, dinv, b)


def kernel(features, edge_index, W1, b1, W2, b2, W3, b3):
    src = edge_index[0]
    dst = edge_index[1]
    pad = E_PAD - E
    r = jnp.arange(pad, dtype=jnp.int32)
    src_p = jnp.concatenate([src, r % N]).reshape(NT * CPT, CHUNK)
    dst_p = jnp.concatenate([dst, N + (r % (N_PAD - N))]).reshape(NT * CPT, CHUNK)

    W3p = jnp.concatenate([W3, jnp.zeros((D_H, D_H - D_OUT), jnp.float32)], axis=1)

    deg_parts = _sc_deg(dst_p)
    dinv, z1 = _tc_first(deg_parts, features, W1)
    acc1 = _sc_agg_h(z1, src_p, dst_p)
    z2 = _tc_mid(acc1, z1, dinv, b1.reshape(1, D_H), W2, D_H)
    acc2 = _sc_agg_h(z2, src_p, dst_p)
    z3 = _tc_mid(acc2, z2, dinv, b2.reshape(1, D_H), W3p, D_H)
    acc3 = _sc_agg_h(z3, src_p, dst_p)
    return _tc_last(acc3, z3, dinv, b3.reshape(1, D_OUT))


# speedup vs baseline: 22.4971x; 1.1464x over previous
import functools

import jax
import jax.numpy as jnp
from jax import lax
from jax.experimental import pallas as pl
from jax.experimental.pallas import tpu as pltpu
from jax.experimental.pallas import tpu_sc as plsc

N = 10000
E = 320000
D_IN = 128
D_H = 128
D_OUT = 64

NC = 2
NS = 16
NT = NC * NS
CHUNK = 128
CPT = 80
IB = 8
E_PAD = NT * CPT * CHUNK
N_PAD = 10240
RPT = N_PAD // NS
DEG_W = 128
BR = 2000

_mesh = plsc.VectorSubcoreMesh(
    core_axis_name="c", subcore_axis_name="s", num_cores=NC, num_subcores=NS)


@functools.partial(
    pl.kernel,
    out_type=jax.ShapeDtypeStruct((NC, N_PAD, DEG_W), jnp.float32),
    mesh=_mesh,
    scratch_types=[
        pltpu.VMEM((IB, CHUNK), jnp.int32),
        pltpu.VMEM((CHUNK, DEG_W), jnp.float32),
        pltpu.VMEM_SHARED((N_PAD, DEG_W), jnp.float32),
        pltpu.SemaphoreType.DMA,
    ],
)
def _sc_deg(dst_hbm, out_hbm, dst_v, buf_v, acc_sh, sem):
    c = lax.axis_index("c")
    s = lax.axis_index("s")
    t = c * NS + s
    zero16 = jnp.zeros((16,), jnp.float32)
    one16 = jnp.full((16,), 1.0, jnp.float32)

    @pl.loop(0, CHUNK)
    def _zero(r):
        for k in range(DEG_W // 16):
            buf_v[r, pl.ds(k * 16, 16)] = zero16

    @pl.loop(0, RPT // CHUNK)
    def _init(k):
        pltpu.sync_copy(buf_v, acc_sh.at[pl.ds(s * RPT + k * CHUNK, CHUNK)])

    @pl.loop(0, CHUNK)
    def _ones(r):
        for k in range(DEG_W // 16):
            buf_v[r, pl.ds(k * 16, 16)] = one16

    plsc.subcore_barrier()

    @pl.loop(0, CPT // IB)
    def _blk(bi):
        pltpu.sync_copy(dst_hbm.at[pl.ds(t * CPT + bi * IB, IB)], dst_v)
        descs = [pltpu.async_copy(buf_v, acc_sh.at[dst_v.at[j]], sem, add=True)
                 for j in range(IB)]
        for d in descs:
            d.wait()

    plsc.subcore_barrier()

    @pl.loop(0, RPT // CHUNK)
    def _wb(k):
        off = s * RPT + k * CHUNK
        pltpu.sync_copy(acc_sh.at[pl.ds(off, CHUNK)], buf_v)
        pltpu.sync_copy(buf_v, out_hbm.at[c, pl.ds(off, CHUNK)])


def _make_sc_agg(D):
    @functools.partial(
        pl.kernel,
        out_type=jax.ShapeDtypeStruct((NC, N_PAD, D), jnp.float32),
        mesh=_mesh,
        scratch_types=[
            pltpu.VMEM((IB, CHUNK), jnp.int32),
            pltpu.VMEM((IB, CHUNK), jnp.int32),
            pltpu.VMEM((CHUNK, D), jnp.float32),
            pltpu.VMEM((CHUNK, D), jnp.float32),
            pltpu.VMEM_SHARED((N_PAD, D), jnp.float32),
            pltpu.SemaphoreType.DMA,
            pltpu.SemaphoreType.DMA,
            pltpu.SemaphoreType.DMA,
            pltpu.SemaphoreType.DMA,
        ],
    )
    def _sc_agg(z_hbm, src_hbm, dst_hbm, out_hbm,
                src_v, dst_v, ra, rb, acc_sh, sa, sb, sca, scb):
        c = lax.axis_index("c")
        s = lax.axis_index("s")
        t = c * NS + s
        zero16 = jnp.zeros((16,), jnp.float32)

        @pl.loop(0, CHUNK)
        def _zero(r):
            for k in range(D // 16):
                ra[r, pl.ds(k * 16, 16)] = zero16

        @pl.loop(0, RPT // CHUNK)
        def _init(k):
            pltpu.sync_copy(ra, acc_sh.at[pl.ds(s * RPT + k * CHUNK, CHUNK)])

        plsc.subcore_barrier()

        bufs = (ra, rb)
        gsems = (sa, sb)
        ssems = (sca, scb)

        @pl.loop(0, CPT // IB)
        def _blk(bi):
            base = t * CPT + bi * IB
            pltpu.sync_copy(src_hbm.at[pl.ds(base, IB)], src_v)
            pltpu.sync_copy(dst_hbm.at[pl.ds(base, IB)], dst_v)
            g = [pltpu.async_copy(z_hbm.at[src_v.at[k]], bufs[k], gsems[k])
                 for k in range(2)]
            s = [None, None]
            for j in range(IB):
                k = j % 2
                g[k].wait()
                s[k] = pltpu.async_copy(
                    bufs[k], acc_sh.at[dst_v.at[j]], ssems[k], add=True)
                if j + 2 < IB:
                    s[k].wait()
                    g[k] = pltpu.async_copy(
                        z_hbm.at[src_v.at[j + 2]], bufs[k], gsems[k])
            s[0].wait()
            s[1].wait()

        plsc.subcore_barrier()

        @pl.loop(0, RPT // CHUNK)
        def _wb(k):
            off = s * RPT + k * CHUNK
            pltpu.sync_copy(acc_sh.at[pl.ds(off, CHUNK)], ra)
            pltpu.sync_copy(ra, out_hbm.at[c, pl.ds(off, CHUNK)])

    return _sc_agg


_sc_agg_h = _make_sc_agg(D_H)


def _tc_first_body(degp_ref, x_ref, w_ref, dinv_ref, z_ref):
    deg = degp_ref[0, :, 0:1] + degp_ref[1, :, 0:1] + 1.0
    dinv = lax.rsqrt(deg)
    dinv_ref[...] = dinv
    xw = jnp.dot(x_ref[...], w_ref[...], preferred_element_type=jnp.float32)
    z_ref[...] = xw * dinv


def _tc_first(deg_parts, features, W1):
    return pl.pallas_call(
        _tc_first_body,
        grid=(N // BR,),
        in_specs=[
            pl.BlockSpec((NC, BR, DEG_W), lambda i: (0, i, 0)),
            pl.BlockSpec((BR, D_IN), lambda i: (i, 0)),
            pl.BlockSpec((D_IN, D_H), lambda i: (0, 0)),
        ],
        out_specs=[
            pl.BlockSpec((BR, 1), lambda i: (i, 0)),
            pl.BlockSpec((BR, D_H), lambda i: (i, 0)),
        ],
        out_shape=[
            jax.ShapeDtypeStruct((N, 1), jnp.float32),
            jax.ShapeDtypeStruct((N, D_H), jnp.float32),
        ],
    )(deg_parts, features, W1)


def _tc_mid_body(acc_ref, z_ref, dinv_ref, b_ref, w_ref, zn_ref):
    dinv = dinv_ref[...]
    x = (acc_ref[0] + acc_ref[1] + z_ref[...]) * dinv + b_ref[...]
    x = jnp.maximum(x, 0.0)
    zn_ref[...] = jnp.dot(x, w_ref[...], preferred_element_type=jnp.float32) * dinv


def _tc_mid(acc_parts, z, dinv, b, W, d_out):
    d_in = z.shape[1]
    return pl.pallas_call(
        _tc_mid_body,
        grid=(N // BR,),
        in_specs=[
            pl.BlockSpec((NC, BR, d_in), lambda i: (0, i, 0)),
            pl.BlockSpec((BR, d_in), lambda i: (i, 0)),
            pl.BlockSpec((BR, 1), lambda i: (i, 0)),
            pl.BlockSpec((1, d_in), lambda i: (0, 0)),
            pl.BlockSpec((d_in, d_out), lambda i: (0, 0)),
        ],
        out_specs=pl.BlockSpec((BR, d_out), lambda i: (i, 0)),
        out_shape=jax.ShapeDtypeStruct((N, d_out), jnp.float32),
    )(acc_parts, z, dinv, b, W)


def _tc_last_body(acc_ref, z_ref, dinv_ref, b_ref, out_ref):
    acc = acc_ref[0, :, 0:D_OUT] + acc_ref[1, :, 0:D_OUT]
    x = (acc + z_ref[:, 0:D_OUT]) * dinv_ref[...] + b_ref[...]
    x = jnp.maximum(x, 0.0)
    m = jnp.max(x, axis=-1, keepdims=True)
    lse = jnp.log(jnp.sum(jnp.exp(x - m), axis=-1, keepdims=True)) + m
    out_ref[...] = x - lse


def _tc_last(acc_parts, z, dinv, b):
    return pl.pallas_call(
        _tc_last_body,
        grid=(N // BR,),
        in_specs=[
            pl.BlockSpec((NC, BR, D_H), lambda i: (0, i, 0)),
            pl.BlockSpec((BR, D_H), lambda i: (i, 0)),
            pl.BlockSpec((BR, 1), lambda i: (i, 0)),
            pl.BlockSpec((1, D_OUT), lambda i: (0, 0)),
        ],
        out_specs=pl.BlockSpec((BR, D_OUT), lambda i: (i, 0)),
        out_shape=jax.ShapeDtypeStruct((N, D_OUT), jnp.float32),
    )(acc_parts, z, dinv, b)


def kernel(features, edge_index, W1, b1, W2, b2, W3, b3):
    src = edge_index[0]
    dst = edge_index[1]
    pad = E_PAD - E
    r = jnp.arange(pad, dtype=jnp.int32)
    src_p = jnp.concatenate([src, r % N]).reshape(NT * CPT, CHUNK)
    dst_p = jnp.concatenate([dst, N + (r % (N_PAD - N))]).reshape(NT * CPT, CHUNK)

    W3p = jnp.concatenate([W3, jnp.zeros((D_H, D_H - D_OUT), jnp.float32)], axis=1)

    deg_parts = _sc_deg(dst_p)
    dinv, z1 = _tc_first(deg_parts, features, W1)
    acc1 = _sc_agg_h(z1, src_p, dst_p)
    z2 = _tc_mid(acc1, z1, dinv, b1.reshape(1, D_H), W2, D_H)
    acc2 = _sc_agg_h(z2, src_p, dst_p)
    z3 = _tc_mid(acc2, z2, dinv, b2.reshape(1, D_H), W3p, D_H)
    acc3 = _sc_agg_h(z3, src_p, dst_p)
    return _tc_last(acc3, z3, dinv, b3.reshape(1, D_OUT))


# speedup vs baseline: 23.8692x; 1.0610x over previous
import functools

import jax
import jax.numpy as jnp
from jax import lax
from jax.experimental import pallas as pl
from jax.experimental.pallas import tpu as pltpu
from jax.experimental.pallas import tpu_sc as plsc

N = 10000
E = 320000
D_IN = 128
D_H = 128
D_OUT = 64

NC = 2
NS = 16
NT = NC * NS
CHUNK = 128
CPT = 80
IB = 8
IBA = 8
E_PAD = NT * CPT * CHUNK
N_PAD = 10240
RPT = N_PAD // NS
DEG_W = 128
BR = 2000

_mesh = plsc.VectorSubcoreMesh(
    core_axis_name="c", subcore_axis_name="s", num_cores=NC, num_subcores=NS)


@functools.partial(
    pl.kernel,
    out_type=jax.ShapeDtypeStruct((NC, N_PAD, DEG_W), jnp.float32),
    mesh=_mesh,
    scratch_types=[
        pltpu.VMEM((IB, CHUNK), jnp.int32),
        pltpu.VMEM((CHUNK, DEG_W), jnp.float32),
        pltpu.VMEM_SHARED((N_PAD, DEG_W), jnp.float32),
        pltpu.SemaphoreType.DMA,
    ],
)
def _sc_deg(dst_hbm, out_hbm, dst_v, buf_v, acc_sh, sem):
    c = lax.axis_index("c")
    s = lax.axis_index("s")
    t = c * NS + s
    zero16 = jnp.zeros((16,), jnp.float32)
    one16 = jnp.full((16,), 1.0, jnp.float32)

    @pl.loop(0, CHUNK)
    def _zero(r):
        for k in range(DEG_W // 16):
            buf_v[r, pl.ds(k * 16, 16)] = zero16

    @pl.loop(0, RPT // CHUNK)
    def _init(k):
        pltpu.sync_copy(buf_v, acc_sh.at[pl.ds(s * RPT + k * CHUNK, CHUNK)])

    @pl.loop(0, CHUNK)
    def _ones(r):
        for k in range(DEG_W // 16):
            buf_v[r, pl.ds(k * 16, 16)] = one16

    plsc.subcore_barrier()

    @pl.loop(0, CPT // IB)
    def _blk(bi):
        pltpu.sync_copy(dst_hbm.at[pl.ds(t * CPT + bi * IB, IB)], dst_v)
        descs = [pltpu.async_copy(buf_v, acc_sh.at[dst_v.at[j]], sem, add=True)
                 for j in range(IB)]
        for d in descs:
            d.wait()

    plsc.subcore_barrier()

    @pl.loop(0, RPT // CHUNK)
    def _wb(k):
        off = s * RPT + k * CHUNK
        pltpu.sync_copy(acc_sh.at[pl.ds(off, CHUNK)], buf_v)
        pltpu.sync_copy(buf_v, out_hbm.at[c, pl.ds(off, CHUNK)])


def _make_sc_agg(D):
    @functools.partial(
        pl.kernel,
        out_type=jax.ShapeDtypeStruct((NC, N_PAD, D), jnp.float32),
        mesh=_mesh,
        scratch_types=[
            pltpu.VMEM((IBA, CHUNK), jnp.int32),
            pltpu.VMEM((IBA, CHUNK), jnp.int32),
            pltpu.VMEM((IBA, CHUNK), jnp.int32),
            pltpu.VMEM((IBA, CHUNK), jnp.int32),
            pltpu.VMEM((CHUNK, D), jnp.float32),
            pltpu.VMEM((CHUNK, D), jnp.float32),
            pltpu.VMEM_SHARED((N_PAD, D), jnp.float32),
            pltpu.SemaphoreType.DMA,
            pltpu.SemaphoreType.DMA,
            pltpu.SemaphoreType.DMA,
            pltpu.SemaphoreType.DMA,
            pltpu.SemaphoreType.DMA,
        ],
    )
    def _sc_agg(z_hbm, src_hbm, dst_hbm, out_hbm,
                src0, dst0, src1, dst1, ra, rb, acc_sh, sa, sb, sca, scb, six):
        c = lax.axis_index("c")
        s = lax.axis_index("s")
        t = c * NS + s
        zero16 = jnp.zeros((16,), jnp.float32)

        @pl.loop(0, CHUNK)
        def _zero(r):
            for k in range(D // 16):
                ra[r, pl.ds(k * 16, 16)] = zero16

        @pl.loop(0, RPT // CHUNK)
        def _init(k):
            pltpu.sync_copy(ra, acc_sh.at[pl.ds(s * RPT + k * CHUNK, CHUNK)])

        plsc.subcore_barrier()

        bufs = (ra, rb)
        gsems = (sa, sb)
        ssems = (sca, scb)
        isets = ((src0, dst0), (src1, dst1))
        NBLK = CPT // IBA

        def _run_block(base, sv, dv, nbase, nsv, ndv, prefetch, drain):
            if drain:
                pltpu.make_async_copy(src_hbm.at[pl.ds(base, IBA)], sv, six).wait()
                pltpu.make_async_copy(dst_hbm.at[pl.ds(base, IBA)], dv, six).wait()
            if prefetch:
                pltpu.async_copy(src_hbm.at[pl.ds(nbase, IBA)], nsv, six)
                pltpu.async_copy(dst_hbm.at[pl.ds(nbase, IBA)], ndv, six)
            g = [pltpu.async_copy(z_hbm.at[sv.at[k]], bufs[k], gsems[k])
                 for k in range(2)]
            s = [None, None]
            for j in range(IBA):
                k = j % 2
                g[k].wait()
                s[k] = pltpu.async_copy(
                    bufs[k], acc_sh.at[dv.at[j]], ssems[k], add=True)
                if j + 2 < IBA:
                    s[k].wait()
                    g[k] = pltpu.async_copy(
                        z_hbm.at[sv.at[j + 2]], bufs[k], gsems[k])
            s[0].wait()
            s[1].wait()

        base0 = t * CPT
        pltpu.sync_copy(src_hbm.at[pl.ds(base0, IBA)], src0)
        pltpu.sync_copy(dst_hbm.at[pl.ds(base0, IBA)], dst0)
        _run_block(base0, src0, dst0, base0 + IBA, src1, dst1,
                   prefetch=True, drain=False)

        @pl.loop(1, NBLK - 1, step=2)
        def _blk(bi):
            for h in range(2):
                sv, dv = isets[(1 + h) % 2]
                nsv, ndv = isets[h]
                base = t * CPT + (bi + h) * IBA
                _run_block(base, sv, dv, base + IBA, nsv, ndv,
                           prefetch=True, drain=True)

        basel = t * CPT + (NBLK - 1) * IBA
        _run_block(basel, src1, dst1, 0, src0, dst0,
                   prefetch=False, drain=True)

        plsc.subcore_barrier()

        @pl.loop(0, RPT // CHUNK)
        def _wb(k):
            off = s * RPT + k * CHUNK
            pltpu.sync_copy(acc_sh.at[pl.ds(off, CHUNK)], ra)
            pltpu.sync_copy(ra, out_hbm.at[c, pl.ds(off, CHUNK)])

    return _sc_agg


_sc_agg_h = _make_sc_agg(D_H)


def _tc_xw_body(x_ref, w_ref, xw_ref):
    xw_ref[...] = jnp.dot(x_ref[...], w_ref[...],
                          preferred_element_type=jnp.float32)


def _tc_xw(features, W1):
    return pl.pallas_call(
        _tc_xw_body,
        grid=(N // BR,),
        in_specs=[
            pl.BlockSpec((BR, D_IN), lambda i: (i, 0)),
            pl.BlockSpec((D_IN, D_H), lambda i: (0, 0)),
        ],
        out_specs=pl.BlockSpec((BR, D_H), lambda i: (i, 0)),
        out_shape=jax.ShapeDtypeStruct((N, D_H), jnp.float32),
    )(features, W1)


def _tc_first_body(degp_ref, xw_ref, dinv_ref, z_ref):
    deg = degp_ref[0, :, 0:1] + degp_ref[1, :, 0:1] + 1.0
    dinv = lax.rsqrt(deg)
    dinv_ref[...] = dinv
    z_ref[...] = xw_ref[...] * dinv


def _tc_first(deg_parts, xw):
    return pl.pallas_call(
        _tc_first_body,
        grid=(N // BR,),
        in_specs=[
            pl.BlockSpec((NC, BR, DEG_W), lambda i: (0, i, 0)),
            pl.BlockSpec((BR, D_H), lambda i: (i, 0)),
        ],
        out_specs=[
            pl.BlockSpec((BR, 1), lambda i: (i, 0)),
            pl.BlockSpec((BR, D_H), lambda i: (i, 0)),
        ],
        out_shape=[
            jax.ShapeDtypeStruct((N, 1), jnp.float32),
            jax.ShapeDtypeStruct((N, D_H), jnp.float32),
        ],
    )(deg_parts, xw)


def _tc_mid_body(acc_ref, z_ref, dinv_ref, b_ref, w_ref, zn_ref):
    dinv = dinv_ref[...]
    x = (acc_ref[0] + acc_ref[1] + z_ref[...]) * dinv + b_ref[...]
    x = jnp.maximum(x, 0.0)
    zn_ref[...] = jnp.dot(x, w_ref[...], preferred_element_type=jnp.float32) * dinv


def _tc_mid(acc_parts, z, dinv, b, W, d_out):
    d_in = z.shape[1]
    return pl.pallas_call(
        _tc_mid_body,
        grid=(N // BR,),
        in_specs=[
            pl.BlockSpec((NC, BR, d_in), lambda i: (0, i, 0)),
            pl.BlockSpec((BR, d_in), lambda i: (i, 0)),
            pl.BlockSpec((BR, 1), lambda i: (i, 0)),
            pl.BlockSpec((1, d_in), lambda i: (0, 0)),
            pl.BlockSpec((d_in, d_out), lambda i: (0, 0)),
        ],
        out_specs=pl.BlockSpec((BR, d_out), lambda i: (i, 0)),
        out_shape=jax.ShapeDtypeStruct((N, d_out), jnp.float32),
    )(acc_parts, z, dinv, b, W)


def _tc_last_body(acc_ref, z_ref, dinv_ref, b_ref, out_ref):
    acc = acc_ref[0, :, 0:D_OUT] + acc_ref[1, :, 0:D_OUT]
    x = (acc + z_ref[:, 0:D_OUT]) * dinv_ref[...] + b_ref[...]
    x = jnp.maximum(x, 0.0)
    m = jnp.max(x, axis=-1, keepdims=True)
    lse = jnp.log(jnp.sum(jnp.exp(x - m), axis=-1, keepdims=True)) + m
    out_ref[...] = x - lse


def _tc_last(acc_parts, z, dinv, b):
    return pl.pallas_call(
        _tc_last_body,
        grid=(N // BR,),
        in_specs=[
            pl.BlockSpec((NC, BR, D_H), lambda i: (0, i, 0)),
            pl.BlockSpec((BR, D_H), lambda i: (i, 0)),
            pl.BlockSpec((BR, 1), lambda i: (i, 0)),
            pl.BlockSpec((1, D_OUT), lambda i: (0, 0)),
        ],
        out_specs=pl.BlockSpec((BR, D_OUT), lambda i: (i, 0)),
        out_shape=jax.ShapeDtypeStruct((N, D_OUT), jnp.float32),
    )(acc_parts, z, dinv, b)


def kernel(features, edge_index, W1, b1, W2, b2, W3, b3):
    src = edge_index[0]
    dst = edge_index[1]
    pad = E_PAD - E
    r = jnp.arange(pad, dtype=jnp.int32)
    src_p = jnp.concatenate([src, r % N]).reshape(NT * CPT, CHUNK)
    dst_p = jnp.concatenate([dst, N + (r % (N_PAD - N))]).reshape(NT * CPT, CHUNK)

    W3p = jnp.concatenate([W3, jnp.zeros((D_H, D_H - D_OUT), jnp.float32)], axis=1)

    xw1 = _tc_xw(features, W1)
    deg_parts = _sc_deg(dst_p)
    dinv, z1 = _tc_first(deg_parts, xw1)
    acc1 = _sc_agg_h(z1, src_p, dst_p)
    z2 = _tc_mid(acc1, z1, dinv, b1.reshape(1, D_H), W2, D_H)
    acc2 = _sc_agg_h(z2, src_p, dst_p)
    z3 = _tc_mid(acc2, z2, dinv, b2.reshape(1, D_H), W3p, D_H)
    acc3 = _sc_agg_h(z3, src_p, dst_p)
    return _tc_last(acc3, z3, dinv, b3.reshape(1, D_OUT))


# speedup vs baseline: 24.5127x; 1.0270x over previous
import functools

import jax
import jax.numpy as jnp
from jax import lax
from jax.experimental import pallas as pl
from jax.experimental.pallas import tpu as pltpu
from jax.experimental.pallas import tpu_sc as plsc

N = 10000
E = 320000
D_IN = 128
D_H = 128
D_OUT = 64

NC = 2
NS = 16
NT = NC * NS
CHUNK = 128
CPT = 80
IB = 8
IBA = 8
E_PAD = NT * CPT * CHUNK
N_PAD = 10240
RPT = N_PAD // NS
DEG_W = 128
BR = 2000

_mesh = plsc.VectorSubcoreMesh(
    core_axis_name="c", subcore_axis_name="s", num_cores=NC, num_subcores=NS)


@functools.partial(
    pl.kernel,
    out_type=jax.ShapeDtypeStruct((NC, N_PAD, DEG_W), jnp.float32),
    mesh=_mesh,
    scratch_types=[
        pltpu.VMEM((IB, CHUNK), jnp.int32),
        pltpu.VMEM((IB, CHUNK), jnp.int32),
        pltpu.VMEM((CHUNK, DEG_W), jnp.float32),
        pltpu.VMEM_SHARED((N_PAD, DEG_W), jnp.float32),
        pltpu.SemaphoreType.DMA,
        pltpu.SemaphoreType.DMA,
    ],
)
def _sc_deg(dst_hbm, out_hbm, dst0, dst1, buf_v, acc_sh, sem, six):
    c = lax.axis_index("c")
    s = lax.axis_index("s")
    t = c * NS + s
    zero16 = jnp.zeros((16,), jnp.float32)
    one16 = jnp.full((16,), 1.0, jnp.float32)
    NBLK = CPT // IB
    isets = (dst0, dst1)

    pltpu.async_copy(dst_hbm.at[pl.ds(t * CPT, IB)], dst0, six)

    @pl.loop(0, CHUNK)
    def _zero(r):
        for k in range(DEG_W // 16):
            buf_v[r, pl.ds(k * 16, 16)] = zero16

    inits = [pltpu.async_copy(buf_v, acc_sh.at[pl.ds(s * RPT + k * CHUNK, CHUNK)],
                              sem)
             for k in range(RPT // CHUNK)]
    for d in inits:
        d.wait()

    @pl.loop(0, CHUNK)
    def _ones(r):
        for k in range(DEG_W // 16):
            buf_v[r, pl.ds(k * 16, 16)] = one16

    plsc.subcore_barrier()

    def _deg_block(base, dv, nbase, ndv, prefetch):
        pltpu.make_async_copy(dst_hbm.at[pl.ds(base, IB)], dv, six).wait()
        if prefetch:
            pltpu.async_copy(dst_hbm.at[pl.ds(nbase, IB)], ndv, six)
        descs = [pltpu.async_copy(buf_v, acc_sh.at[dv.at[j]], sem, add=True)
                 for j in range(IB)]
        for d in descs:
            d.wait()

    base0 = t * CPT
    _deg_block(base0, dst0, base0 + IB, dst1, True)

    @pl.loop(1, NBLK - 1, step=2)
    def _blk(bi):
        for h in range(2):
            base = t * CPT + (bi + h) * IB
            _deg_block(base, isets[(1 + h) % 2], base + IB, isets[h], True)

    _deg_block(t * CPT + (NBLK - 1) * IB, dst1, 0, dst0, False)

    plsc.subcore_barrier()

    @pl.loop(0, RPT // CHUNK)
    def _wb(k):
        off = s * RPT + k * CHUNK
        pltpu.sync_copy(acc_sh.at[pl.ds(off, CHUNK)], buf_v)
        pltpu.sync_copy(buf_v, out_hbm.at[c, pl.ds(off, CHUNK)])


def _make_sc_agg(D):
    @functools.partial(
        pl.kernel,
        out_type=jax.ShapeDtypeStruct((NC, N_PAD, D), jnp.float32),
        mesh=_mesh,
        scratch_types=[
            pltpu.VMEM((IBA, CHUNK), jnp.int32),
            pltpu.VMEM((IBA, CHUNK), jnp.int32),
            pltpu.VMEM((IBA, CHUNK), jnp.int32),
            pltpu.VMEM((IBA, CHUNK), jnp.int32),
            pltpu.VMEM((CHUNK, D), jnp.float32),
            pltpu.VMEM((CHUNK, D), jnp.float32),
            pltpu.VMEM_SHARED((N_PAD, D), jnp.float32),
            pltpu.SemaphoreType.DMA,
            pltpu.SemaphoreType.DMA,
            pltpu.SemaphoreType.DMA,
            pltpu.SemaphoreType.DMA,
            pltpu.SemaphoreType.DMA,
        ],
    )
    def _sc_agg(z_hbm, src_hbm, dst_hbm, out_hbm,
                src0, dst0, src1, dst1, ra, rb, acc_sh, sa, sb, sca, scb, six):
        c = lax.axis_index("c")
        s = lax.axis_index("s")
        t = c * NS + s
        zero16 = jnp.zeros((16,), jnp.float32)

        base0 = t * CPT
        pltpu.async_copy(src_hbm.at[pl.ds(base0, IBA)], src0, six)
        pltpu.async_copy(dst_hbm.at[pl.ds(base0, IBA)], dst0, six)

        @pl.loop(0, CHUNK)
        def _zero(r):
            for k in range(D // 16):
                ra[r, pl.ds(k * 16, 16)] = zero16

        inits = [pltpu.async_copy(
                     ra, acc_sh.at[pl.ds(s * RPT + k * CHUNK, CHUNK)], sca)
                 for k in range(RPT // CHUNK)]
        for d in inits:
            d.wait()

        plsc.subcore_barrier()

        bufs = (ra, rb)
        gsems = (sa, sb)
        ssems = (sca, scb)
        isets = ((src0, dst0), (src1, dst1))
        NBLK = CPT // IBA

        def _run_block(base, sv, dv, nbase, nsv, ndv, prefetch, drain):
            if drain:
                pltpu.make_async_copy(src_hbm.at[pl.ds(base, IBA)], sv, six).wait()
                pltpu.make_async_copy(dst_hbm.at[pl.ds(base, IBA)], dv, six).wait()
            if prefetch:
                pltpu.async_copy(src_hbm.at[pl.ds(nbase, IBA)], nsv, six)
                pltpu.async_copy(dst_hbm.at[pl.ds(nbase, IBA)], ndv, six)
            g = [pltpu.async_copy(z_hbm.at[sv.at[k]], bufs[k], gsems[k])
                 for k in range(2)]
            s = [None, None]
            for j in range(IBA):
                k = j % 2
                g[k].wait()
                s[k] = pltpu.async_copy(
                    bufs[k], acc_sh.at[dv.at[j]], ssems[k], add=True)
                if j + 2 < IBA:
                    s[k].wait()
                    g[k] = pltpu.async_copy(
                        z_hbm.at[sv.at[j + 2]], bufs[k], gsems[k])
            s[0].wait()
            s[1].wait()

        _run_block(base0, src0, dst0, base0 + IBA, src1, dst1,
                   prefetch=True, drain=True)

        @pl.loop(1, NBLK - 1, step=2)
        def _blk(bi):
            for h in range(2):
                sv, dv = isets[(1 + h) % 2]
                nsv, ndv = isets[h]
                base = t * CPT + (bi + h) * IBA
                _run_block(base, sv, dv, base + IBA, nsv, ndv,
                           prefetch=True, drain=True)

        basel = t * CPT + (NBLK - 1) * IBA
        _run_block(basel, src1, dst1, 0, src0, dst0,
                   prefetch=False, drain=True)

        plsc.subcore_barrier()

        wdesc = [None, None]
        for k in range(RPT // CHUNK):
            buf, wsem = (ra, sa) if k % 2 == 0 else (rb, sb)
            if wdesc[k % 2] is not None:
                wdesc[k % 2].wait()
            off = s * RPT + k * CHUNK
            pltpu.sync_copy(acc_sh.at[pl.ds(off, CHUNK)], buf)
            wdesc[k % 2] = pltpu.async_copy(buf, out_hbm.at[c, pl.ds(off, CHUNK)], wsem)
        for d in wdesc:
            if d is not None:
                d.wait()

    return _sc_agg


_sc_agg_h = _make_sc_agg(D_H)


def _tc_xw_body(x_ref, w_ref, xw_ref):
    xw_ref[...] = jnp.dot(x_ref[...], w_ref[...],
                          preferred_element_type=jnp.float32)


def _tc_xw(features, W1):
    return pl.pallas_call(
        _tc_xw_body,
        grid=(N // BR,),
        in_specs=[
            pl.BlockSpec((BR, D_IN), lambda i: (i, 0)),
            pl.BlockSpec((D_IN, D_H), lambda i: (0, 0)),
        ],
        out_specs=pl.BlockSpec((BR, D_H), lambda i: (i, 0)),
        out_shape=jax.ShapeDtypeStruct((N, D_H), jnp.float32),
    )(features, W1)


def _tc_first_body(degp_ref, xw_ref, dinv_ref, z_ref):
    deg = degp_ref[0, :, 0:1] + degp_ref[1, :, 0:1] + 1.0
    dinv = lax.rsqrt(deg)
    dinv_ref[...] = dinv
    z_ref[...] = xw_ref[...] * dinv


def _tc_first(deg_parts, xw):
    return pl.pallas_call(
        _tc_first_body,
        grid=(N // BR,),
        in_specs=[
            pl.BlockSpec((NC, BR, DEG_W), lambda i: (0, i, 0)),
            pl.BlockSpec((BR, D_H), lambda i: (i, 0)),
        ],
        out_specs=[
            pl.BlockSpec((BR, 1), lambda i: (i, 0)),
            pl.BlockSpec((BR, D_H), lambda i: (i, 0)),
        ],
        out_shape=[
            jax.ShapeDtypeStruct((N, 1), jnp.float32),
            jax.ShapeDtypeStruct((N, D_H), jnp.float32),
        ],
    )(deg_parts, xw)


def _tc_mid_body(acc_ref, z_ref, dinv_ref, b_ref, w_ref, zn_ref):
    dinv = dinv_ref[...]
    x = (acc_ref[0] + acc_ref[1] + z_ref[...]) * dinv + b_ref[...]
    x = jnp.maximum(x, 0.0)
    zn_ref[...] = jnp.dot(x, w_ref[...], preferred_element_type=jnp.float32) * dinv


def _tc_mid(acc_parts, z, dinv, b, W, d_out):
    d_in = z.shape[1]
    return pl.pallas_call(
        _tc_mid_body,
        grid=(N // BR,),
        in_specs=[
            pl.BlockSpec((NC, BR, d_in), lambda i: (0, i, 0)),
            pl.BlockSpec((BR, d_in), lambda i: (i, 0)),
            pl.BlockSpec((BR, 1), lambda i: (i, 0)),
            pl.BlockSpec((1, d_in), lambda i: (0, 0)),
            pl.BlockSpec((d_in, d_out), lambda i: (0, 0)),
        ],
        out_specs=pl.BlockSpec((BR, d_out), lambda i: (i, 0)),
        out_shape=jax.ShapeDtypeStruct((N, d_out), jnp.float32),
    )(acc_parts, z, dinv, b, W)


def _tc_last_body(acc_ref, z_ref, dinv_ref, b_ref, out_ref):
    acc = acc_ref[0, :, 0:D_OUT] + acc_ref[1, :, 0:D_OUT]
    x = (acc + z_ref[:, 0:D_OUT]) * dinv_ref[...] + b_ref[...]
    x = jnp.maximum(x, 0.0)
    m = jnp.max(x, axis=-1, keepdims=True)
    lse = jnp.log(jnp.sum(jnp.exp(x - m), axis=-1, keepdims=True)) + m
    out_ref[...] = x - lse


def _tc_last(acc_parts, z, dinv, b):
    return pl.pallas_call(
        _tc_last_body,
        grid=(N // BR,),
        in_specs=[
            pl.BlockSpec((NC, BR, D_H), lambda i: (0, i, 0)),
            pl.BlockSpec((BR, D_H), lambda i: (i, 0)),
            pl.BlockSpec((BR, 1), lambda i: (i, 0)),
            pl.BlockSpec((1, D_OUT), lambda i: (0, 0)),
        ],
        out_specs=pl.BlockSpec((BR, D_OUT), lambda i: (i, 0)),
        out_shape=jax.ShapeDtypeStruct((N, D_OUT), jnp.float32),
    )(acc_parts, z, dinv, b)


def kernel(features, edge_index, W1, b1, W2, b2, W3, b3):
    src = edge_index[0]
    dst = edge_index[1]
    pad = E_PAD - E
    r = jnp.arange(pad, dtype=jnp.int32)
    src_p = jnp.concatenate([src, r % N]).reshape(NT * CPT, CHUNK)
    dst_p = jnp.concatenate([dst, N + (r % (N_PAD - N))]).reshape(NT * CPT, CHUNK)

    W3p = jnp.concatenate([W3, jnp.zeros((D_H, D_H - D_OUT), jnp.float32)], axis=1)

    xw1 = _tc_xw(features, W1)
    deg_parts = _sc_deg(dst_p)
    dinv, z1 = _tc_first(deg_parts, xw1)
    acc1 = _sc_agg_h(z1, src_p, dst_p)
    z2 = _tc_mid(acc1, z1, dinv, b1.reshape(1, D_H), W2, D_H)
    acc2 = _sc_agg_h(z2, src_p, dst_p)
    z3 = _tc_mid(acc2, z2, dinv, b2.reshape(1, D_H), W3p, D_H)
    acc3 = _sc_agg_h(z3, src_p, dst_p)
    return _tc_last(acc3, z3, dinv, b3.reshape(1, D_OUT))


# speedup vs baseline: 24.5637x; 1.0021x over previous
import functools

import jax
import jax.numpy as jnp
from jax import lax
from jax.experimental import pallas as pl
from jax.experimental.pallas import tpu as pltpu
from jax.experimental.pallas import tpu_sc as plsc

N = 10000
E = 320000
D_IN = 128
D_H = 128
D_OUT = 64

NC = 2
NS = 16
NT = NC * NS
CHUNK = 128
CPT = 80
IB = 8
IBA = 8
E_PAD = NT * CPT * CHUNK
N_PAD = 10240
RPT = N_PAD // NS
DEG_W = 128
BR = 2000

_mesh = plsc.VectorSubcoreMesh(
    core_axis_name="c", subcore_axis_name="s", num_cores=NC, num_subcores=NS)


@functools.partial(
    pl.kernel,
    out_type=jax.ShapeDtypeStruct((NC, N_PAD, DEG_W), jnp.float32),
    mesh=_mesh,
    scratch_types=[
        pltpu.VMEM((IB, CHUNK), jnp.int32),
        pltpu.VMEM((IB, CHUNK), jnp.int32),
        pltpu.VMEM((CHUNK, DEG_W), jnp.float32),
        pltpu.VMEM_SHARED((N_PAD, DEG_W), jnp.float32),
        pltpu.SemaphoreType.DMA,
        pltpu.SemaphoreType.DMA,
    ],
)
def _sc_deg(dst_hbm, out_hbm, dst0, dst1, buf_v, acc_sh, sem, six):
    c = lax.axis_index("c")
    s = lax.axis_index("s")
    t = c * NS + s
    zero16 = jnp.zeros((16,), jnp.float32)
    one16 = jnp.full((16,), 1.0, jnp.float32)
    NBLK = CPT // IB
    isets = (dst0, dst1)

    pltpu.async_copy(dst_hbm.at[pl.ds(t * CPT, IB)], dst0, six)

    @pl.loop(0, CHUNK)
    def _zero(r):
        for k in range(DEG_W // 16):
            buf_v[r, pl.ds(k * 16, 16)] = zero16

    inits = [pltpu.async_copy(buf_v, acc_sh.at[pl.ds(s * RPT + k * CHUNK, CHUNK)],
                              sem)
             for k in range(RPT // CHUNK)]
    for d in inits:
        d.wait()

    @pl.loop(0, CHUNK)
    def _ones(r):
        for k in range(DEG_W // 16):
            buf_v[r, pl.ds(k * 16, 16)] = one16

    plsc.subcore_barrier()

    def _deg_block(base, dv, nbase, ndv, prefetch):
        pltpu.make_async_copy(dst_hbm.at[pl.ds(base, IB)], dv, six).wait()
        if prefetch:
            pltpu.async_copy(dst_hbm.at[pl.ds(nbase, IB)], ndv, six)
        descs = [pltpu.async_copy(buf_v, acc_sh.at[dv.at[j]], sem, add=True)
                 for j in range(IB)]
        for d in descs:
            d.wait()

    base0 = t * CPT
    _deg_block(base0, dst0, base0 + IB, dst1, True)

    @pl.loop(1, NBLK - 1, step=2)
    def _blk(bi):
        for h in range(2):
            base = t * CPT + (bi + h) * IB
            _deg_block(base, isets[(1 + h) % 2], base + IB, isets[h], True)

    _deg_block(t * CPT + (NBLK - 1) * IB, dst1, 0, dst0, False)

    plsc.subcore_barrier()

    @pl.loop(0, RPT // CHUNK)
    def _wb(k):
        off = s * RPT + k * CHUNK
        pltpu.sync_copy(acc_sh.at[pl.ds(off, CHUNK)], buf_v)
        pltpu.sync_copy(buf_v, out_hbm.at[c, pl.ds(off, CHUNK)])


def _make_sc_agg(D):
    @functools.partial(
        pl.kernel,
        out_type=jax.ShapeDtypeStruct((NC, N_PAD, D), jnp.float32),
        mesh=_mesh,
        scratch_types=[
            pltpu.VMEM((IBA, CHUNK), jnp.int32),
            pltpu.VMEM((IBA, CHUNK), jnp.int32),
            pltpu.VMEM((IBA, CHUNK), jnp.int32),
            pltpu.VMEM((IBA, CHUNK), jnp.int32),
            pltpu.VMEM((CHUNK, D), jnp.float32),
            pltpu.VMEM((CHUNK, D), jnp.float32),
            pltpu.VMEM_SHARED((N_PAD, D), jnp.float32),
            pltpu.SemaphoreType.DMA,
            pltpu.SemaphoreType.DMA,
            pltpu.SemaphoreType.DMA,
            pltpu.SemaphoreType.DMA,
            pltpu.SemaphoreType.DMA,
        ],
    )
    def _sc_agg(z_hbm, src_hbm, dst_hbm, out_hbm,
                src0, dst0, src1, dst1, ra, rb, acc_sh, sa, sb, sca, scb, six):
        c = lax.axis_index("c")
        s = lax.axis_index("s")
        t = c * NS + s
        zero16 = jnp.zeros((16,), jnp.float32)

        base0 = t * CPT
        pltpu.async_copy(src_hbm.at[pl.ds(base0, IBA)], src0, six)
        pltpu.async_copy(dst_hbm.at[pl.ds(base0, IBA)], dst0, six)

        @pl.loop(0, CHUNK)
        def _zero(r):
            for k in range(D // 16):
                ra[r, pl.ds(k * 16, 16)] = zero16

        inits = [pltpu.async_copy(
                     ra, acc_sh.at[pl.ds(s * RPT + k * CHUNK, CHUNK)], sca)
                 for k in range(RPT // CHUNK)]
        for d in inits:
            d.wait()

        plsc.subcore_barrier()

        bufs = (ra, rb)
        gsems = (sa, sb)
        ssems = (sca, scb)
        isets = ((src0, dst0), (src1, dst1))
        NBLK = CPT // IBA

        def _run_block(base, sv, dv, nbase, nsv, ndv, prefetch, drain):
            if drain:
                pltpu.make_async_copy(src_hbm.at[pl.ds(base, IBA)], sv, six).wait()
                pltpu.make_async_copy(dst_hbm.at[pl.ds(base, IBA)], dv, six).wait()
            if prefetch:
                pltpu.async_copy(src_hbm.at[pl.ds(nbase, IBA)], nsv, six)
                pltpu.async_copy(dst_hbm.at[pl.ds(nbase, IBA)], ndv, six)
            g = [pltpu.async_copy(z_hbm.at[sv.at[k]], bufs[k], gsems[k])
                 for k in range(2)]
            s = [None, None]
            for j in range(IBA):
                k = j % 2
                g[k].wait()
                s[k] = pltpu.async_copy(
                    bufs[k], acc_sh.at[dv.at[j]], ssems[k], add=True)
                if j + 2 < IBA:
                    s[k].wait()
                    g[k] = pltpu.async_copy(
                        z_hbm.at[sv.at[j + 2]], bufs[k], gsems[k])
            s[0].wait()
            s[1].wait()

        _run_block(base0, src0, dst0, base0 + IBA, src1, dst1,
                   prefetch=True, drain=True)

        @pl.loop(1, NBLK - 1, step=2)
        def _blk(bi):
            for h in range(2):
                sv, dv = isets[(1 + h) % 2]
                nsv, ndv = isets[h]
                base = t * CPT + (bi + h) * IBA
                _run_block(base, sv, dv, base + IBA, nsv, ndv,
                           prefetch=True, drain=True)

        basel = t * CPT + (NBLK - 1) * IBA
        _run_block(basel, src1, dst1, 0, src0, dst0,
                   prefetch=False, drain=True)

        plsc.subcore_barrier()

        wdesc = [None, None]
        for k in range(RPT // CHUNK):
            buf, wsem = (ra, sa) if k % 2 == 0 else (rb, sb)
            if wdesc[k % 2] is not None:
                wdesc[k % 2].wait()
            off = s * RPT + k * CHUNK
            pltpu.sync_copy(acc_sh.at[pl.ds(off, CHUNK)], buf)
            wdesc[k % 2] = pltpu.async_copy(buf, out_hbm.at[c, pl.ds(off, CHUNK)], wsem)
        for d in wdesc:
            if d is not None:
                d.wait()

    return _sc_agg


_sc_agg_h = _make_sc_agg(D_H)


def _tc_xw_body(x_ref, w_ref, xw_ref):
    xw_ref[...] = jnp.dot(x_ref[...], w_ref[...],
                          preferred_element_type=jnp.float32)


def _tc_xw(features, W1):
    return pl.pallas_call(
        _tc_xw_body,
        grid=(N // BR,),
        in_specs=[
            pl.BlockSpec((BR, D_IN), lambda i: (i, 0)),
            pl.BlockSpec((D_IN, D_H), lambda i: (0, 0)),
        ],
        out_specs=pl.BlockSpec((BR, D_H), lambda i: (i, 0)),
        out_shape=jax.ShapeDtypeStruct((N, D_H), jnp.float32),
    )(features, W1)


def _tc_first_body(degp_ref, x_ref, w_ref, dinv_ref, z_ref):
    deg = degp_ref[0, :, 0:1] + degp_ref[1, :, 0:1] + 1.0
    dinv = lax.rsqrt(deg)
    dinv_ref[...] = dinv
    xw = jnp.dot(x_ref[...], w_ref[...], preferred_element_type=jnp.float32)
    z_ref[...] = xw * dinv


def _tc_first(deg_parts, features, W1):
    return pl.pallas_call(
        _tc_first_body,
        grid=(N // BR,),
        in_specs=[
            pl.BlockSpec((NC, BR, DEG_W), lambda i: (0, i, 0)),
            pl.BlockSpec((BR, D_IN), lambda i: (i, 0)),
            pl.BlockSpec((D_IN, D_H), lambda i: (0, 0)),
        ],
        out_specs=[
            pl.BlockSpec((BR, 1), lambda i: (i, 0)),
            pl.BlockSpec((BR, D_H), lambda i: (i, 0)),
        ],
        out_shape=[
            jax.ShapeDtypeStruct((N, 1), jnp.float32),
            jax.ShapeDtypeStruct((N, D_H), jnp.float32),
        ],
    )(deg_parts, features, W1)


def _tc_mid_body(acc_ref, z_ref, dinv_ref, b_ref, w_ref, zn_ref):
    dinv = dinv_ref[...]
    x = (acc_ref[0] + acc_ref[1] + z_ref[...]) * dinv + b_ref[...]
    x = jnp.maximum(x, 0.0)
    zn_ref[...] = jnp.dot(x, w_ref[...], preferred_element_type=jnp.float32) * dinv


def _tc_mid(acc_parts, z, dinv, b, W, d_out):
    d_in = z.shape[1]
    return pl.pallas_call(
        _tc_mid_body,
        grid=(N // BR,),
        in_specs=[
            pl.BlockSpec((NC, BR, d_in), lambda i: (0, i, 0)),
            pl.BlockSpec((BR, d_in), lambda i: (i, 0)),
            pl.BlockSpec((BR, 1), lambda i: (i, 0)),
            pl.BlockSpec((1, d_in), lambda i: (0, 0)),
            pl.BlockSpec((d_in, d_out), lambda i: (0, 0)),
        ],
        out_specs=pl.BlockSpec((BR, d_out), lambda i: (i, 0)),
        out_shape=jax.ShapeDtypeStruct((N, d_out), jnp.float32),
    )(acc_parts, z, dinv, b, W)


def _tc_last_body(acc_ref, z_ref, dinv_ref, b_ref, out_ref):
    acc = acc_ref[0, :, 0:D_OUT] + acc_ref[1, :, 0:D_OUT]
    x = (acc + z_ref[:, 0:D_OUT]) * dinv_ref[...] + b_ref[...]
    x = jnp.maximum(x, 0.0)
    m = jnp.max(x, axis=-1, keepdims=True)
    lse = jnp.log(jnp.sum(jnp.exp(x - m), axis=-1, keepdims=True)) + m
    out_ref[...] = x - lse


def _tc_last(acc_parts, z, dinv, b):
    return pl.pallas_call(
        _tc_last_body,
        grid=(N // BR,),
        in_specs=[
            pl.BlockSpec((NC, BR, D_H), lambda i: (0, i, 0)),
            pl.BlockSpec((BR, D_H), lambda i: (i, 0)),
            pl.BlockSpec((BR, 1), lambda i: (i, 0)),
            pl.BlockSpec((1, D_OUT), lambda i: (0, 0)),
        ],
        out_specs=pl.BlockSpec((BR, D_OUT), lambda i: (i, 0)),
        out_shape=jax.ShapeDtypeStruct((N, D_OUT), jnp.float32),
    )(acc_parts, z, dinv, b)


def kernel(features, edge_index, W1, b1, W2, b2, W3, b3):
    src = edge_index[0]
    dst = edge_index[1]
    pad = E_PAD - E
    r = jnp.arange(pad, dtype=jnp.int32)
    src_p = jnp.concatenate([src, r % N]).reshape(NT * CPT, CHUNK)
    dst_p = jnp.concatenate([dst, N + (r % (N_PAD - N))]).reshape(NT * CPT, CHUNK)

    W3p = jnp.concatenate([W3, jnp.zeros((D_H, D_H - D_OUT), jnp.float32)], axis=1)

    deg_parts = _sc_deg(dst_p)
    dinv, z1 = _tc_first(deg_parts, features, W1)
    acc1 = _sc_agg_h(z1, src_p, dst_p)
    z2 = _tc_mid(acc1, z1, dinv, b1.reshape(1, D_H), W2, D_H)
    acc2 = _sc_agg_h(z2, src_p, dst_p)
    z3 = _tc_mid(acc2, z2, dinv, b2.reshape(1, D_H), W3p, D_H)
    acc3 = _sc_agg_h(z3, src_p, dst_p)
    return _tc_last(acc3, z3, dinv, b3.reshape(1, D_OUT))
